# Initial kernel scaffold; baseline (speedup 1.0000x reference)
import jax
import jax.numpy as jnp
from jax.experimental import pallas as pl


def kernel(node_feats, W_ih, W_hh, b_ih, b_hh):
    raise NotImplementedError("write your pallas kernel here")


# speedup vs baseline: 6.4904x; 6.4904x over previous
import jax
import jax.numpy as jnp
from jax.experimental import pallas as pl
from jax.experimental.pallas import tpu as pltpu

N = 10000
C = 256
BLK = 1000


def _gru_block_kernel(x_ref, wihT_ref, whhT_ref, bih_ref, bhh_ref, out_ref,
                      h_scratch, gi_scratch):
    pi = pl.program_id(0)

    @pl.when(pi == 0)
    def _init():
        h_scratch[...] = jnp.zeros_like(h_scratch)

    gi_scratch[...] = jnp.dot(x_ref[...], wihT_ref[...],
                              preferred_element_type=jnp.float32) + bih_ref[...]

    whhT = whhT_ref[...]
    bhh = bhh_ref[...]

    def step(t, h):
        gi = gi_scratch[pl.ds(t, 1), :]
        gh = jnp.dot(h, whhT, preferred_element_type=jnp.float32) + bhh
        i_r = gi[:, 0:C]
        i_z = gi[:, C:2 * C]
        i_n = gi[:, 2 * C:3 * C]
        h_r = gh[:, 0:C]
        h_z = gh[:, C:2 * C]
        h_n = gh[:, 2 * C:3 * C]
        r = jax.nn.sigmoid(i_r + h_r)
        z = jax.nn.sigmoid(i_z + h_z)
        n = jnp.tanh(i_n + r * h_n)
        return (1.0 - z) * n + z * h

    h = jax.lax.fori_loop(0, BLK, step, h_scratch[...])
    h_scratch[...] = h

    @pl.when(pi == pl.num_programs(0) - 1)
    def _out():
        out_ref[...] = h


def kernel(node_feats, W_ih, W_hh, b_ih, b_hh):
    wihT = W_ih.T
    whhT = W_hh.T
    bih = b_ih[None, :]
    bhh = b_hh[None, :]

    grid = (N // BLK,)
    out = pl.pallas_call(
        _gru_block_kernel,
        grid=grid,
        in_specs=[
            pl.BlockSpec((BLK, C), lambda i: (i, 0)),
            pl.BlockSpec((C, 3 * C), lambda i: (0, 0)),
            pl.BlockSpec((C, 3 * C), lambda i: (0, 0)),
            pl.BlockSpec((1, 3 * C), lambda i: (0, 0)),
            pl.BlockSpec((1, 3 * C), lambda i: (0, 0)),
        ],
        out_specs=pl.BlockSpec((1, C), lambda i: (0, 0)),
        out_shape=jax.ShapeDtypeStruct((1, C), jnp.float32),
        scratch_shapes=[pltpu.VMEM((1, C), jnp.float32),
                        pltpu.VMEM((BLK, 3 * C), jnp.float32)],
    )(node_feats, wihT, whhT, bih, bhh)
    return out


# speedup vs baseline: 7.3967x; 1.1396x over previous
import jax
import jax.numpy as jnp
from jax.experimental import pallas as pl
from jax.experimental.pallas import tpu as pltpu

N = 10000
C = 256
BLK = 1000


def _gru_block_kernel(x_ref, wihT_ref, whhT_ref, bih_ref, bhh_ref, out_ref,
                      h_scratch, gi_scratch):
    pi = pl.program_id(0)

    @pl.when(pi == 0)
    def _init():
        h_scratch[...] = jnp.zeros_like(h_scratch)

    gi_scratch[...] = jnp.dot(x_ref[...], wihT_ref[...],
                              preferred_element_type=jnp.float32) + bih_ref[...]

    whhT = whhT_ref[...].astype(jnp.bfloat16)
    bhh = bhh_ref[...]

    def step(t, h):
        gi = gi_scratch[pl.ds(t, 1), :]
        gh = jnp.dot(h.astype(jnp.bfloat16), whhT,
                     preferred_element_type=jnp.float32) + bhh
        i_r = gi[:, 0:C]
        i_z = gi[:, C:2 * C]
        i_n = gi[:, 2 * C:3 * C]
        h_r = gh[:, 0:C]
        h_z = gh[:, C:2 * C]
        h_n = gh[:, 2 * C:3 * C]
        r = jax.nn.sigmoid(i_r + h_r)
        z = jax.nn.sigmoid(i_z + h_z)
        n = jnp.tanh(i_n + r * h_n)
        return (1.0 - z) * n + z * h

    h = jax.lax.fori_loop(0, BLK, step, h_scratch[...], unroll=4)
    h_scratch[...] = h

    @pl.when(pi == pl.num_programs(0) - 1)
    def _out():
        out_ref[...] = h


def kernel(node_feats, W_ih, W_hh, b_ih, b_hh):
    wihT = W_ih.T
    whhT = W_hh.T
    bih = b_ih[None, :]
    bhh = b_hh[None, :]

    grid = (N // BLK,)
    out = pl.pallas_call(
        _gru_block_kernel,
        grid=grid,
        in_specs=[
            pl.BlockSpec((BLK, C), lambda i: (i, 0)),
            pl.BlockSpec((C, 3 * C), lambda i: (0, 0)),
            pl.BlockSpec((C, 3 * C), lambda i: (0, 0)),
            pl.BlockSpec((1, 3 * C), lambda i: (0, 0)),
            pl.BlockSpec((1, 3 * C), lambda i: (0, 0)),
        ],
        out_specs=pl.BlockSpec((1, C), lambda i: (0, 0)),
        out_shape=jax.ShapeDtypeStruct((1, C), jnp.float32),
        scratch_shapes=[pltpu.VMEM((1, C), jnp.float32),
                        pltpu.VMEM((BLK, 3 * C), jnp.float32)],
    )(node_feats, wihT, whhT, bih, bhh)
    return out


# speedup vs baseline: 7.6011x; 1.0276x over previous
import jax
import jax.numpy as jnp
from jax.experimental import pallas as pl
from jax.experimental.pallas import tpu as pltpu

N = 10000
C = 256
BLK = 1000


def _gru_block_kernel(x_ref, wihT_ref, whhT_ref, bih_ref, bhhn_ref, out_ref,
                      h_scratch, gi_scratch):
    pi = pl.program_id(0)

    @pl.when(pi == 0)
    def _init():
        h_scratch[...] = jnp.zeros_like(h_scratch)

    gi_scratch[...] = jnp.dot(x_ref[...], wihT_ref[...],
                              preferred_element_type=jnp.float32) + bih_ref[...]

    whhT = whhT_ref[...].astype(jnp.bfloat16)
    bhhn = bhhn_ref[...]

    def step(t, h):
        gi = gi_scratch[pl.ds(t, 1), :]
        gh = jnp.dot(h.astype(jnp.bfloat16), whhT,
                     preferred_element_type=jnp.float32)
        i_r = gi[:, 0:C]
        i_z = gi[:, C:2 * C]
        i_n = gi[:, 2 * C:3 * C]
        h_r = gh[:, 0:C]
        h_z = gh[:, C:2 * C]
        h_n = gh[:, 2 * C:3 * C] + bhhn
        r = jax.nn.sigmoid(i_r + h_r)
        z = jax.nn.sigmoid(i_z + h_z)
        n = jnp.tanh(i_n + r * h_n)
        return (1.0 - z) * n + z * h

    h = jax.lax.fori_loop(0, BLK, step, h_scratch[...], unroll=8)
    h_scratch[...] = h

    @pl.when(pi == pl.num_programs(0) - 1)
    def _out():
        out_ref[...] = h


def kernel(node_feats, W_ih, W_hh, b_ih, b_hh):
    wihT = W_ih.T
    whhT = W_hh.T
    bih = jnp.concatenate([b_ih[:2 * C] + b_hh[:2 * C], b_ih[2 * C:]])[None, :]
    bhhn = b_hh[2 * C:][None, :]

    grid = (N // BLK,)
    out = pl.pallas_call(
        _gru_block_kernel,
        grid=grid,
        in_specs=[
            pl.BlockSpec((BLK, C), lambda i: (i, 0)),
            pl.BlockSpec((C, 3 * C), lambda i: (0, 0)),
            pl.BlockSpec((C, 3 * C), lambda i: (0, 0)),
            pl.BlockSpec((1, 3 * C), lambda i: (0, 0)),
            pl.BlockSpec((1, C), lambda i: (0, 0)),
        ],
        out_specs=pl.BlockSpec((1, C), lambda i: (0, 0)),
        out_shape=jax.ShapeDtypeStruct((1, C), jnp.float32),
        scratch_shapes=[pltpu.VMEM((1, C), jnp.float32),
                        pltpu.VMEM((BLK, 3 * C), jnp.float32)],
    )(node_feats, wihT, whhT, bih, bhhn)
    return out
